# baseline (speedup 1.0000x reference)
import functools

import jax
import jax.numpy as jnp
from jax import lax
from jax.experimental import pallas as pl
from jax.experimental.pallas import tpu as pltpu
from jax.experimental.pallas import tpu_sc as plsc

_D = 768
_B = 16384
_NC = 2
_NS = 16
_NW = _NC * _NS
_BPW = _B // _NW
_C = 32
_NBUF = 3
_NCHUNK = _BPW // _C


def _embed_body(ids_hbm, table_hbm, out_hbm, idx_v, buf0, buf1, buf2,
                table_sh, *sems):
    bufs = (buf0, buf1, buf2)
    sid = lax.axis_index("s")
    wid = sid * _NC + lax.axis_index("c")
    base = wid * _BPW
    pltpu.sync_copy(ids_hbm.at[pl.ds(base, _BPW)], idx_v)
    r0 = sid * 64
    @pl.when(sid < _NS - 1)
    def _():
        pltpu.sync_copy(table_hbm.at[pl.ds(r0 * _D, 64 * _D)],
                        table_sh.at[pl.ds(r0 * _D, 64 * _D)])
    @pl.when(sid == _NS - 1)
    def _():
        pltpu.sync_copy(table_hbm.at[pl.ds(944 * _D, 57 * _D)],
                        table_sh.at[pl.ds(944 * _D, 57 * _D)])
    plsc.subcore_barrier()
    rsem = sems[0]
    wsems = sems[1:]
    whandles = [None] * _NCHUNK
    for g in range(_NCHUNK):
        b = g % _NBUF
        rh = []
        for k in range(_C // 16):
            vec = idx_v[pl.ds(g * _C + 16 * k, 16)]
            for j in range(16):
                i = 16 * k + j
                row = vec[j]
                rh.append(pltpu.async_copy(
                    table_sh.at[pl.ds(row * _D, _D)],
                    bufs[b].at[pl.ds(i * _D, _D)], rsem))
        for h in rh:
            h.wait()
        if g >= _NBUF:
            whandles[g - _NBUF].wait()
        whandles[g] = pltpu.async_copy(
            bufs[b], out_hbm.at[pl.ds((base + g * _C) * _D, _C * _D)],
            wsems[g % _NBUF])
    for g in range(_NCHUNK - _NBUF, _NCHUNK):
        whandles[g].wait()


@jax.jit
def _embed(ids, table):
    mesh = plsc.VectorSubcoreMesh(core_axis_name="c", subcore_axis_name="s")
    f = functools.partial(
        pl.kernel,
        mesh=mesh,
        out_type=jax.ShapeDtypeStruct((_B * _D,), jnp.float32),
        scratch_types=[
            pltpu.VMEM((_BPW,), jnp.int32),
            pltpu.VMEM((_C * _D,), jnp.float32),
            pltpu.VMEM((_C * _D,), jnp.float32),
            pltpu.VMEM((_C * _D,), jnp.float32),
            pltpu.VMEM_SHARED((1008 * _D,), jnp.float32),
        ] + [pltpu.SemaphoreType.DMA] * (1 + _NBUF),
    )(_embed_body)
    return f(ids, table.reshape(-1))


def kernel(input_ids, table):
    ids = input_ids.astype(jnp.int32)
    return _embed(ids, table).reshape(_B, _D)
